# baseline (speedup 1.0000x reference)
import jax
import jax.numpy as jnp
from jax.experimental import pallas as pl


_B, _C, _H, _W = 32, 768, 32, 32
_F = 384
_HW = _H * _W


def _pos_body(row_ref, col_ref, out_ref):
    w_idx = jax.lax.broadcasted_iota(jnp.int32, (_W, _HW), 0)
    q_idx = jax.lax.broadcasted_iota(jnp.int32, (_W, _HW), 1)
    col_pat = (q_idx % _W == w_idx).astype(jnp.float32)
    row_pat = (q_idx // _W == w_idx).astype(jnp.float32)

    top = jax.lax.dot_general(
        col_ref[...], col_pat, (((0,), (0,)), ((), ())),
        preferred_element_type=jnp.float32)
    bot = jax.lax.dot_general(
        row_ref[...], row_pat, (((0,), (0,)), ((), ())),
        preferred_element_type=jnp.float32)
    out_ref[0, :_F, :] = top
    out_ref[0, _F:, :] = bot


def kernel(x, row_embed, col_embed):
    B, C, H, W = x.shape
    out = pl.pallas_call(
        _pos_body,
        grid=(B,),
        in_specs=[
            pl.BlockSpec((_H, _F), lambda b: (0, 0)),
            pl.BlockSpec((_W, _F), lambda b: (0, 0)),
        ],
        out_specs=pl.BlockSpec((1, C, _HW), lambda b: (b, 0, 0)),
        out_shape=jax.ShapeDtypeStruct((B, C, _HW), jnp.float32),
    )(row_embed, col_embed)
    return out.reshape(B, C, H, W)


# speedup vs baseline: 1.0172x; 1.0172x over previous
import jax
import jax.numpy as jnp
from jax.experimental import pallas as pl
from jax.experimental.pallas import tpu as pltpu


_B, _C, _H, _W = 32, 768, 32, 32
_F = 384
_HW = _H * _W


def _pos_body(row_ref, col_ref, out_ref, scratch, sem):
    w_idx = jax.lax.broadcasted_iota(jnp.int32, (_W, _HW), 0)
    q_idx = jax.lax.broadcasted_iota(jnp.int32, (_W, _HW), 1)
    col_pat = (q_idx % _W == w_idx).astype(jnp.float32)
    row_pat = (q_idx // _W == w_idx).astype(jnp.float32)

    scratch[:_F, :] = jax.lax.dot_general(
        col_ref[...], col_pat, (((0,), (0,)), ((), ())),
        preferred_element_type=jnp.float32)
    scratch[_F:, :] = jax.lax.dot_general(
        row_ref[...], row_pat, (((0,), (0,)), ((), ())),
        preferred_element_type=jnp.float32)

    copies = [pltpu.make_async_copy(scratch, out_ref.at[b], sem)
              for b in range(_B)]
    for c in copies:
        c.start()
    for c in copies:
        c.wait()


def kernel(x, row_embed, col_embed):
    B, C, H, W = x.shape
    out = pl.pallas_call(
        _pos_body,
        in_specs=[
            pl.BlockSpec(memory_space=pltpu.VMEM),
            pl.BlockSpec(memory_space=pltpu.VMEM),
        ],
        out_specs=pl.BlockSpec(memory_space=pl.ANY),
        out_shape=jax.ShapeDtypeStruct((B, C, _HW), jnp.float32),
        scratch_shapes=[
            pltpu.VMEM((C, _HW), jnp.float32),
            pltpu.SemaphoreType.DMA,
        ],
    )(row_embed[:H], col_embed[:W])
    return out.reshape(B, C, H, W)
